# Initial kernel scaffold; baseline (speedup 1.0000x reference)
import jax
import jax.numpy as jnp
from jax.experimental import pallas as pl


def kernel(oldx, X, Y):
    raise NotImplementedError("write your pallas kernel here")


# speedup vs baseline: 4.1883x; 4.1883x over previous
import functools
import math

import jax
import jax.numpy as jnp
from jax import lax
from jax.experimental import pallas as pl
from jax.experimental.pallas import tpu as pltpu
from jax.experimental.pallas import tpu_sc as plsc

LA = 9
WA = 1 << LA
TBA = 256
TBB = 256


def _phase_a_body(x_ref, xh_ref, yh_ref, y_ref, node_ref):
    x = x_ref[...]
    s = lax.dot_general(x, xh_ref[...], (((1,), (1,)), ((), ())),
                        preferred_element_type=jnp.float32,
                        precision=lax.Precision.HIGHEST)
    iota = lax.broadcasted_iota(jnp.int32, (TBA, WA), 1)
    node = jnp.zeros((TBA, 1), jnp.int32)
    acc = jnp.zeros((TBA, WA), jnp.float32)
    for _ in range(LA):
        onehot = iota == node
        lam = jnp.sum(jnp.where(onehot, s, 0.0), axis=1, keepdims=True)
        acc = acc + jnp.where(onehot, lam, 0.0)
        node = 2 * node + 1 + (lam > 0.0).astype(jnp.int32)
    y_ref[...] = lax.dot_general(acc, yh_ref[...], (((1,), (0,)), ((), ())),
                                 preferred_element_type=jnp.float32,
                                 precision=lax.Precision.HIGHEST)
    node_ref[...] = node.reshape(1, 1, TBA)


def _phase_a(x, xh, yh):
    b, f = x.shape
    grid = (b // TBA,)
    return pl.pallas_call(
        _phase_a_body,
        grid=grid,
        in_specs=[
            pl.BlockSpec((TBA, f), lambda t: (t, 0)),
            pl.BlockSpec((WA, f), lambda t: (0, 0)),
            pl.BlockSpec((WA, f), lambda t: (0, 0)),
        ],
        out_specs=[
            pl.BlockSpec((TBA, f), lambda t: (t, 0)),
            pl.BlockSpec((1, 1, TBA), lambda t: (t, 0, 0)),
        ],
        out_shape=[
            jax.ShapeDtypeStruct((b, f), jnp.float32),
            jax.ShapeDtypeStruct((b // TBA, 1, TBA), jnp.int32),
        ],
    )(x, xh, yh)


def _upd_body(x_ref, xn_ref, yn_ref, yin_ref, nin_ref, yout_ref, nout_ref):
    lam = jnp.sum(x_ref[...] * xn_ref[...], axis=1, keepdims=True)
    yout_ref[...] = yin_ref[...] + lam * yn_ref[...]
    nout_ref[...] = (2 * nin_ref[...] + 1
                     + (lam > 0.0).astype(jnp.int32).reshape(1, 1, TBB))


def _update(x, xn, yn, yin, nin):
    b, f = x.shape
    grid = (b // TBB,)
    row = pl.BlockSpec((TBB, f), lambda t: (t, 0))
    nspec = pl.BlockSpec((1, 1, TBB), lambda t: (t, 0, 0))
    return pl.pallas_call(
        _upd_body,
        grid=grid,
        in_specs=[row, row, row, row, nspec],
        out_specs=[row, nspec],
        out_shape=[
            jax.ShapeDtypeStruct((b, f), jnp.float32),
            jax.ShapeDtypeStruct((b // TBB, 1, TBB), jnp.int32),
        ],
    )(x, xn, yn, yin, nin)


def _sc_gather_pair(xtab, ytab, idx):
    b = idx.shape[0]
    f = xtab.shape[1]
    info = plsc.get_sparse_core_info()
    nc, ns = info.num_cores, info.num_subcores
    nw = nc * ns
    bpw = b // nw
    ck = 8
    nchunk = bpw // ck
    mesh = plsc.VectorSubcoreMesh(core_axis_name="c", subcore_axis_name="s")

    @functools.partial(
        pl.kernel,
        mesh=mesh,
        out_type=[jax.ShapeDtypeStruct((b, f), jnp.float32),
                  jax.ShapeDtypeStruct((b, f), jnp.float32)],
        scratch_types=[
            pltpu.VMEM((bpw,), jnp.int32),
            pltpu.VMEM((ck, f), jnp.float32),
            pltpu.VMEM((ck, f), jnp.float32),
            pltpu.SemaphoreType.DMA,
            pltpu.SemaphoreType.DMA,
        ],
    )
    def k(xt_hbm, yt_hbm, idx_hbm, outx_hbm, outy_hbm,
          idx_v, bufx, bufy, sx, sy):
        wid = lax.axis_index("s") * nc + lax.axis_index("c")
        base = wid * bpw
        pltpu.sync_copy(idx_hbm.at[pl.ds(base, bpw)], idx_v)

        def body(c, carry):
            off = c * ck
            cx = pltpu.async_copy(xt_hbm.at[idx_v.at[pl.ds(off, ck)]], bufx, sx)
            cy = pltpu.async_copy(yt_hbm.at[idx_v.at[pl.ds(off, ck)]], bufy, sy)
            cx.wait()
            pltpu.sync_copy(bufx, outx_hbm.at[pl.ds(base + off, ck)])
            cy.wait()
            pltpu.sync_copy(bufy, outy_hbm.at[pl.ds(base + off, ck)])
            return carry

        lax.fori_loop(0, nchunk, body, 0)

    return k(xtab, ytab, idx)


def kernel(oldx, X, Y):
    f = X.shape[-1]
    depth = int(math.floor(math.log2(f)))
    x = oldx.reshape(-1, f)
    b = x.shape[0]

    y, node3 = _phase_a(x, X[:WA], Y[:WA])
    for _ in range(LA, depth):
        node = node3.reshape(b)
        xn, yn = _sc_gather_pair(X, Y, node)
        y, node3 = _update(x, xn, yn, y, node3)
    return y.reshape(oldx.shape)
